# baseline (speedup 1.0000x reference)
import functools

import jax
import jax.numpy as jnp
from jax import lax
from jax.experimental import pallas as pl
from jax.experimental.pallas import tpu as pltpu
from jax.experimental.pallas import tpu_sc as plsc

EMBED_DIM = 600
NUM_WORKERS = 32
ROWS = 100
CHUNKS = 64


def _embed_gather(idx2d, table):
    n_chunks_total = idx2d.shape[0]
    mesh = plsc.VectorSubcoreMesh(core_axis_name="c", subcore_axis_name="s")

    @functools.partial(
        pl.kernel,
        mesh=mesh,
        compiler_params=pltpu.CompilerParams(use_tc_tiling_on_sc=False),
        out_type=jax.ShapeDtypeStruct((n_chunks_total, ROWS, EMBED_DIM), jnp.float32),
        scratch_types=[
            pltpu.VMEM((CHUNKS, ROWS), jnp.int32),
            pltpu.VMEM((2, ROWS, EMBED_DIM), jnp.float32),
            pltpu.SemaphoreType.DMA((2,)),
        ],
    )
    def k(idx_hbm, table_hbm, out_hbm, idx_v, rows_v, sems):
        wid = lax.axis_index("s") * 2 + lax.axis_index("c")
        cbase = wid * CHUNKS
        pltpu.sync_copy(idx_hbm.at[pl.ds(cbase, CHUNKS)], idx_v)
        pltpu.async_copy(table_hbm.at[idx_v.at[0]], rows_v.at[0], sems.at[0])

        def body(g, carry):
            b = lax.rem(g, 2)
            nb = lax.rem(g + 1, 2)

            @pl.when(g + 1 < CHUNKS)
            def _():
                pltpu.async_copy(
                    table_hbm.at[idx_v.at[g + 1]], rows_v.at[nb], sems.at[nb]
                )

            pltpu.make_async_copy(
                table_hbm.at[idx_v.at[g]], rows_v.at[b], sems.at[b]
            ).wait()
            pltpu.sync_copy(rows_v.at[b], out_hbm.at[cbase + g])
            return carry

        lax.fori_loop(0, CHUNKS, body, 0)

    return k(idx2d, table)


def kernel(batch, table):
    B, L = batch.shape
    idx2d = batch.reshape(NUM_WORKERS * CHUNKS, ROWS)
    out = _embed_gather(idx2d, table)
    return out.reshape(B, L, EMBED_DIM)


# speedup vs baseline: 1.3676x; 1.3676x over previous
import functools

import jax
import jax.numpy as jnp
from jax import lax
from jax.experimental import pallas as pl
from jax.experimental.pallas import tpu as pltpu
from jax.experimental.pallas import tpu_sc as plsc

EMBED_DIM = 600
PAD_DIM = 640
NUM_WORKERS = 32
ROWS = 64
CHUNKS = 100


def _embed_gather(idx3d, table_pad):
    mesh = plsc.VectorSubcoreMesh(core_axis_name="c", subcore_axis_name="s")

    @functools.partial(
        pl.kernel,
        mesh=mesh,
        out_type=jax.ShapeDtypeStruct(
            (NUM_WORKERS, CHUNKS, ROWS, PAD_DIM), jnp.float32
        ),
        scratch_types=[
            pltpu.VMEM((CHUNKS, ROWS), jnp.int32),
            pltpu.VMEM((2, ROWS, PAD_DIM), jnp.float32),
            pltpu.SemaphoreType.DMA((2,)),
        ],
    )
    def k(idx_hbm, table_hbm, out_hbm, idx_v, rows_v, sems):
        wid = lax.axis_index("s") * 2 + lax.axis_index("c")
        pltpu.sync_copy(idx_hbm.at[wid], idx_v)
        pltpu.async_copy(table_hbm.at[idx_v.at[0]], rows_v.at[0], sems.at[0])

        def body(g, carry):
            b = lax.rem(g, 2)
            nb = lax.rem(g + 1, 2)

            @pl.when(g + 1 < CHUNKS)
            def _():
                pltpu.async_copy(
                    table_hbm.at[idx_v.at[g + 1]], rows_v.at[nb], sems.at[nb]
                )

            pltpu.make_async_copy(
                table_hbm.at[idx_v.at[g]], rows_v.at[b], sems.at[b]
            ).wait()
            pltpu.sync_copy(rows_v.at[b], out_hbm.at[wid, g])
            return carry

        lax.fori_loop(0, CHUNKS, body, 0)

    return k(idx3d, table_pad)


def kernel(batch, table):
    B, L = batch.shape
    idx3d = batch.reshape(NUM_WORKERS, CHUNKS, ROWS)
    table_pad = jnp.pad(table, ((0, 0), (0, PAD_DIM - EMBED_DIM)))
    out = _embed_gather(idx3d, table_pad)
    return out.reshape(B, L, PAD_DIM)[:, :, :EMBED_DIM]


# speedup vs baseline: 2.5035x; 1.8307x over previous
import functools

import jax
import jax.numpy as jnp
from jax import lax
from jax.experimental import pallas as pl
from jax.experimental.pallas import tpu as pltpu
from jax.experimental.pallas import tpu_sc as plsc

EMBED_DIM = 600
MAIN_DIM = 512
TAIL_DIM = 88
TAIL_PAD = 128
NUM_WORKERS = 32
ROWS = 64
CHUNKS = 100


def _embed_gather(idx3d, table, tail):
    mesh = plsc.VectorSubcoreMesh(core_axis_name="c", subcore_axis_name="s")

    @functools.partial(
        pl.kernel,
        mesh=mesh,
        out_type=jax.ShapeDtypeStruct(
            (NUM_WORKERS, CHUNKS, ROWS, EMBED_DIM), jnp.float32
        ),
        scratch_types=[
            pltpu.VMEM((CHUNKS, ROWS), jnp.int32),
            pltpu.VMEM((2, ROWS, EMBED_DIM), jnp.float32),
            pltpu.VMEM((2, ROWS, TAIL_PAD), jnp.float32),
            pltpu.SemaphoreType.DMA((2,)),
            pltpu.SemaphoreType.DMA((2,)),
        ],
    )
    def k(idx_hbm, table_hbm, tail_hbm, out_hbm, idx_v, stage_v, tail_v, sems, sems_t):
        wid = lax.axis_index("s") * 2 + lax.axis_index("c")
        pltpu.sync_copy(idx_hbm.at[wid], idx_v)

        def start_gathers(g, b):
            pltpu.async_copy(
                table_hbm.at[idx_v.at[g], pl.ds(0, MAIN_DIM)],
                stage_v.at[b, :, pl.ds(0, MAIN_DIM)],
                sems.at[b],
            )
            pltpu.async_copy(tail_hbm.at[idx_v.at[g]], tail_v.at[b], sems_t.at[b])

        start_gathers(0, 0)

        def body(g, carry):
            b = lax.rem(g, 2)
            nb = lax.rem(g + 1, 2)

            @pl.when(g + 1 < CHUNKS)
            def _():
                start_gathers(g + 1, nb)

            pltpu.make_async_copy(
                table_hbm.at[idx_v.at[g], pl.ds(0, MAIN_DIM)],
                stage_v.at[b, :, pl.ds(0, MAIN_DIM)],
                sems.at[b],
            ).wait()
            pltpu.make_async_copy(
                tail_hbm.at[idx_v.at[g]], tail_v.at[b], sems_t.at[b]
            ).wait()

            def merge_row(r, c):
                for i in range(5):
                    stage_v[b, r, pl.ds(MAIN_DIM + i * 16, 16)] = tail_v[
                        b, r, pl.ds(i * 16, 16)
                    ]
                stage_v[b, r, pl.ds(EMBED_DIM - 16, 16)] = tail_v[
                    b, r, pl.ds(TAIL_DIM - 16, 16)
                ]
                return c

            lax.fori_loop(0, ROWS, merge_row, 0)

            pltpu.sync_copy(stage_v.at[b], out_hbm.at[wid, g])
            return carry

        lax.fori_loop(0, CHUNKS, body, 0)

    return k(idx3d, table, tail)


def kernel(batch, table):
    B, L = batch.shape
    idx3d = batch.reshape(NUM_WORKERS, CHUNKS, ROWS)
    tail = jnp.pad(table[:, MAIN_DIM:], ((0, 0), (0, TAIL_PAD - TAIL_DIM)))
    out = _embed_gather(idx3d, table, tail)
    return out.reshape(B, L, EMBED_DIM)
